# baseline (speedup 1.0000x reference)
import functools

import jax
import jax.numpy as jnp
from jax import lax
from jax.experimental import pallas as pl
from jax.experimental.pallas import tpu as pltpu
from jax.experimental.pallas import tpu_sc as plsc

N_NODES = 10000
N_EDGES = 320000
D_FEAT = 128

NC = 2
NS = 16
NW = NC * NS
C = 64
K = 160
EPT = K * C
NBUF = 5
ROWS_PT = 632
N_PAD = NS * ROWS_PT
LAST_ROWS = N_NODES - (NS - 1) * ROWS_PT
IDX_MASK = (1 << 14) - 1


def _sc_graph_conv(x, ew_p):
    mesh = plsc.VectorSubcoreMesh(core_axis_name="c", subcore_axis_name="s",
                                  num_cores=NC, num_subcores=NS)

    @functools.partial(
        pl.kernel,
        out_type=(jax.ShapeDtypeStruct((N_NODES, D_FEAT), jnp.float32),) * 2,
        mesh=mesh,
        compiler_params=pltpu.CompilerParams(use_tc_tiling_on_sc=False),
        scratch_types=(
            [pltpu.VMEM((2, C), jnp.int32) for _ in range(NBUF)]
            + [pltpu.VMEM((C,), jnp.int32) for _ in range(NBUF)]
            + [pltpu.VMEM((C,), jnp.int32) for _ in range(NBUF)]
            + [pltpu.VMEM((C, D_FEAT), jnp.float32) for _ in range(NBUF)]
            + [pltpu.VMEM_SHARED((N_PAD, D_FEAT), jnp.float32)]
            + [pltpu.SemaphoreType.DMA for _ in range(3 * NBUF)]
        ),
    )
    def body(x_hbm, ew_hbm, p0, p1, *rest):
        ering = rest[0:NBUF]
        sring = rest[NBUF:2 * NBUF]
        tring = rest[2 * NBUF:3 * NBUF]
        bufs = rest[3 * NBUF:4 * NBUF]
        acc = rest[4 * NBUF]
        esem = rest[4 * NBUF + 1:4 * NBUF + 1 + NBUF]
        gsem = rest[4 * NBUF + 1 + NBUF:4 * NBUF + 1 + 2 * NBUF]
        ssem = rest[4 * NBUF + 1 + 2 * NBUF:4 * NBUF + 1 + 3 * NBUF]
        cid = lax.axis_index("c")
        sid = lax.axis_index("s")
        wid = cid * NS + sid

        zbuf = bufs[0]

        def zrow(i, carry):
            for f in range(D_FEAT // 16):
                zbuf[i, pl.ds(f * 16, 16)] = jnp.zeros((16,), jnp.float32)
            return carry

        lax.fori_loop(0, C, zrow, 0)
        base = sid * ROWS_PT
        nfull = ROWS_PT // C
        rem = ROWS_PT - nfull * C
        for q in range(nfull):
            pltpu.sync_copy(zbuf, acc.at[pl.ds(base + q * C, C)])
        if rem:
            pltpu.sync_copy(zbuf.at[pl.ds(0, rem)],
                            acc.at[pl.ds(base + nfull * C, rem)])
        plsc.subcore_barrier()

        def e_start(jj, b):
            pltpu.async_copy(ew_hbm.at[wid, jj], ering[b], esem[b])

        def e_wait(jj, b):
            pltpu.make_async_copy(ew_hbm.at[wid, jj], ering[b],
                                  esem[b]).wait()

        def unpack_idx(jj, b):
            @plsc.parallel_loop(0, C // 16)
            def u(g):
                p = ering[b][0, pl.ds(g * 16, 16)]
                sring[b][pl.ds(g * 16, 16)] = p & IDX_MASK
                tring[b][pl.ds(g * 16, 16)] = (p >> 14) & IDX_MASK

        def g_start(jj, b):
            pltpu.async_copy(x_hbm.at[sring[b]], bufs[b], gsem[b])

        def g_wait(jj, b):
            pltpu.make_async_copy(x_hbm.at[sring[b]], bufs[b],
                                  gsem[b]).wait()

        def s_start(jj, b):
            pltpu.async_copy(bufs[b], acc.at[tring[b]], ssem[b], add=True)

        def s_wait(jj, b):
            pltpu.make_async_copy(bufs[b], acc.at[tring[b]], ssem[b]).wait()

        def scale(jj, b):
            buf = bufs[b]
            wrow = ering[b]

            @plsc.parallel_loop(0, C // 16, unroll=2)
            def grp(g):
                wv = lax.bitcast_convert_type(
                    wrow[1, pl.ds(g * 16, 16)], jnp.float32)
                for e in range(16):
                    ws = wv[e]
                    r = g * 16 + e
                    for f in range(D_FEAT // 16):
                        buf[r, pl.ds(f * 16, 16)] = (
                            buf[r, pl.ds(f * 16, 16)] * ws)

        for jj in range(4):
            e_start(jj, jj)
        for jj in (0, 1):
            e_wait(jj, jj)
            unpack_idx(jj, jj)
            g_start(jj, jj)

        def step(i, carry):
            j = i * NBUF
            for b in range(NBUF):
                jj = j + b
                b2 = (b + 2) % NBUF
                b4 = (b + 4) % NBUF

                @pl.when(jj + 4 < K)
                def _():
                    e_start(jj + 4, b4)

                @pl.when(jj >= 3)
                def _():
                    s_wait(jj - 3, b2)

                @pl.when(jj + 2 < K)
                def _():
                    e_wait(jj + 2, b2)
                    unpack_idx(jj + 2, b2)
                    g_start(jj + 2, b2)

                g_wait(jj, b)
                scale(jj, b)
                s_start(jj, b)
            return carry

        lax.fori_loop(0, K // NBUF, step, 0)
        for jj in range(K - 3, K):
            s_wait(jj, jj % NBUF)

        plsc.subcore_barrier()

        for out_ref, my_cid in ((p0, 0), (p1, 1)):
            @pl.when(jnp.logical_and(cid == my_cid, sid < NS - 1))
            def _(out_ref=out_ref):
                pltpu.sync_copy(acc.at[pl.ds(base, ROWS_PT)],
                                out_ref.at[pl.ds(base, ROWS_PT)])

            @pl.when(jnp.logical_and(cid == my_cid, sid == NS - 1))
            def _(out_ref=out_ref):
                pltpu.sync_copy(acc.at[pl.ds(base, LAST_ROWS)],
                                out_ref.at[pl.ds(base, LAST_ROWS)])

    return body(x, ew_p)


def _tc_add(a, b):
    def add_body(a_ref, b_ref, o_ref):
        o_ref[...] = a_ref[...] + b_ref[...]

    return pl.pallas_call(
        add_body,
        out_shape=jax.ShapeDtypeStruct((N_NODES, D_FEAT), jnp.float32),
        grid=(10,),
        in_specs=[pl.BlockSpec((N_NODES // 10, D_FEAT), lambda i: (i, 0))] * 2,
        out_specs=pl.BlockSpec((N_NODES // 10, D_FEAT), lambda i: (i, 0)),
    )(a, b)


def kernel(input, eidx, enorm, esgn):
    sidx = eidx[0].astype(jnp.int32)
    tidx = eidx[1].astype(jnp.int32)
    w = enorm * esgn
    pad = NW * EPT - N_EDGES
    pad_nodes = jnp.arange(pad, dtype=jnp.int32) % N_NODES
    sidx_p = jnp.concatenate([sidx, pad_nodes])
    tidx_p = jnp.concatenate([tidx, pad_nodes])
    w_p = jnp.concatenate([w, jnp.zeros((pad,), jnp.float32)])
    packed = (sidx_p | (tidx_p << 14)).reshape(NW, K, 1, C)
    wbits = lax.bitcast_convert_type(w_p, jnp.int32).reshape(NW, K, 1, C)
    ew_p = jnp.concatenate([packed, wbits], axis=2)
    p0, p1 = _sc_graph_conv(input, ew_p)
    return _tc_add(p0, p1)


# speedup vs baseline: 1.1266x; 1.1266x over previous
import functools

import jax
import jax.numpy as jnp
from jax import lax
from jax.experimental import pallas as pl
from jax.experimental.pallas import tpu as pltpu
from jax.experimental.pallas import tpu_sc as plsc

N_NODES = 10000
N_EDGES = 320000
D_FEAT = 128
DH = D_FEAT // 2

NC = 2
NS = 16
C = 80
K = 250
EPT = K * C
NBUF = 5
NFH = DH // 16
ROWS_PT = 640
N_PAD = NS * ROWS_PT
LAST_ROWS = N_NODES - (NS - 1) * ROWS_PT


def _sc_graph_conv(xlo, xhi, sidx_p, tidx_p, w_p):
    mesh = plsc.VectorSubcoreMesh(core_axis_name="c", subcore_axis_name="s",
                                  num_cores=NC, num_subcores=NS)

    @functools.partial(
        pl.kernel,
        out_type=jax.ShapeDtypeStruct((N_NODES, D_FEAT), jnp.float32),
        mesh=mesh,
        compiler_params=pltpu.CompilerParams(use_tc_tiling_on_sc=False),
        scratch_types=(
            [
                pltpu.VMEM((K, C), jnp.int32),
                pltpu.VMEM((K, C), jnp.int32),
                pltpu.VMEM((K, C), jnp.float32),
            ]
            + [pltpu.VMEM((C, DH), jnp.float32) for _ in range(NBUF)]
            + [pltpu.VMEM_SHARED((N_PAD, DH), jnp.float32)]
            + [pltpu.SemaphoreType.DMA for _ in range(2 * NBUF)]
        ),
    )
    def body(xlo_hbm, xhi_hbm, sidx_hbm, tidx_hbm, w_hbm, out,
             sidx_v, tidx_v, w_v, b0, b1, b2, b3, b4, acc,
             g0, g1, g2, g3, g4, s0, s1, s2, s3, s4):
        bufs = (b0, b1, b2, b3, b4)
        gsem = (g0, g1, g2, g3, g4)
        ssem = (s0, s1, s2, s3, s4)
        cid = lax.axis_index("c")
        sid = lax.axis_index("s")

        pltpu.sync_copy(sidx_hbm.at[sid], sidx_v)
        pltpu.sync_copy(tidx_hbm.at[sid], tidx_v)
        pltpu.sync_copy(w_hbm.at[sid], w_v)

        zbuf = bufs[0]

        def zrow(i, carry):
            for f in range(NFH):
                zbuf[i, pl.ds(f * 16, 16)] = jnp.zeros((16,), jnp.float32)
            return carry

        lax.fori_loop(0, C, zrow, 0)
        base = sid * ROWS_PT
        nfull = ROWS_PT // C
        rem = ROWS_PT - nfull * C
        for q in range(nfull):
            pltpu.sync_copy(zbuf, acc.at[pl.ds(base + q * C, C)])
        if rem:
            pltpu.sync_copy(zbuf.at[pl.ds(0, rem)],
                            acc.at[pl.ds(base + nfull * C, rem)])
        plsc.subcore_barrier()

        def g_start(jj, b):
            @pl.when(cid == 0)
            def _():
                pltpu.async_copy(xlo_hbm.at[sidx_v.at[jj]], bufs[b], gsem[b])

            @pl.when(cid == 1)
            def _():
                pltpu.async_copy(xhi_hbm.at[sidx_v.at[jj]], bufs[b], gsem[b])

        def g_wait(jj, b):
            pltpu.make_async_copy(xlo_hbm.at[sidx_v.at[jj]], bufs[b],
                                  gsem[b]).wait()

        def s_start(jj, b):
            pltpu.async_copy(bufs[b], acc.at[tidx_v.at[jj]], ssem[b],
                             add=True)

        def s_wait(jj, b):
            pltpu.make_async_copy(bufs[b], acc.at[tidx_v.at[jj]],
                                  ssem[b]).wait()

        def scale(jj, b):
            buf = bufs[b]

            @plsc.parallel_loop(0, C // 16, unroll=2)
            def grp(g):
                wv = w_v[jj, pl.ds(g * 16, 16)]
                for e in range(16):
                    ws = wv[e]
                    r = g * 16 + e
                    for f in range(NFH):
                        buf[r, pl.ds(f * 16, 16)] = (
                            buf[r, pl.ds(f * 16, 16)] * ws)

        g_start(0, 0)
        g_start(1, 1)
        g_start(2, 2)

        def step(i, carry):
            j = i * NBUF
            for b in range(NBUF):
                jj = j + b
                b3 = (b + 3) % NBUF

                @pl.when(jj >= NBUF - 3)
                def _():
                    s_wait(jj - (NBUF - 3), b3)

                @pl.when(jj + 3 < K)
                def _():
                    g_start(jj + 3, b3)

                g_wait(jj, b)
                scale(jj, b)
                s_start(jj, b)
            return carry

        lax.fori_loop(0, K // NBUF, step, 0)
        for jj in range(K - NBUF + 3, K):
            s_wait(jj, jj % NBUF)

        plsc.subcore_barrier()

        col = cid * DH

        @pl.when(sid < NS - 1)
        def _():
            pltpu.sync_copy(acc.at[pl.ds(base, ROWS_PT)],
                            out.at[pl.ds(base, ROWS_PT), pl.ds(col, DH)])

        @pl.when(sid == NS - 1)
        def _():
            pltpu.sync_copy(acc.at[pl.ds(base, LAST_ROWS)],
                            out.at[pl.ds(base, LAST_ROWS), pl.ds(col, DH)])

    return body(xlo, xhi, sidx_p, tidx_p, w_p)


def kernel(input, eidx, enorm, esgn):
    sidx_p = eidx[0].astype(jnp.int32).reshape(NS, K, C)
    tidx_p = eidx[1].astype(jnp.int32).reshape(NS, K, C)
    w_p = (enorm * esgn).reshape(NS, K, C)
    return _sc_graph_conv(input[:, :DH], input[:, DH:],
                          sidx_p, tidx_p, w_p)
